# baseline (speedup 1.0000x reference)
import functools

import jax
import jax.numpy as jnp
from jax import lax
from jax.experimental import pallas as pl
from jax.experimental.pallas import tpu as pltpu
from jax.experimental.pallas import tpu_sc as plsc

NN = 100000
EE = 1600000
DD = 128
CC = 32
HH = 16

NC = 2
NS = 16
BLK = 1024
NP = 98 * BLK
SN = NP // NS
ZB = SN // 8

EPT = EE // NS
EPW = EE // (NC * NS)
BD = 2000
BW = 2000
BP = 400

_mesh = plsc.VectorSubcoreMesh(
    core_axis_name="c", subcore_axis_name="s", num_cores=NC, num_subcores=NS)
_sc_params = pltpu.CompilerParams(needs_layout_passes=False,
                                  use_tc_tiling_on_sc=False)

_f32 = jnp.float32
_i32 = jnp.int32


def _rsqrt16(d):
    x = d + 1e-12
    i = lax.bitcast_convert_type(x, _i32)
    i = 0x5F3759DF - lax.shift_right_logical(i, 1)
    y = lax.bitcast_convert_type(i, _f32)
    for _ in range(3):
        y = y * (1.5 - 0.5 * x * y * y)
    return jnp.where(d > 0, y, 0.0)


def _eprep_body(src_hbm, dst_hbm, ea_hbm, w_hbm,
                dinv_v, p1_v, p2_v, p3_v, p4_v, acc_sh):
    c = lax.axis_index("c")
    s = lax.axis_index("s")
    wid = c * NS + s
    stripe = pl.ds(s * SN, SN)

    dz = dinv_v.at[pl.ds(0, SN)]

    def zrow(i, _):
        dz[pl.ds(i * 16, 16)] = jnp.zeros((16,), _f32)
        return 0
    lax.fori_loop(0, SN // 16, zrow, 0)
    pltpu.sync_copy(dz, acc_sh.at[stripe])
    plsc.subcore_barrier()

    def chunk1(j, _):
        base = s * EPT + j * BD
        pltpu.sync_copy(dst_hbm.at[pl.ds(base, BD)], p1_v)
        pltpu.sync_copy(ea_hbm.at[pl.ds(base, BD)], p2_v)
        pltpu.sync_copy(p2_v, acc_sh.at[p1_v], add=True)
        return 0
    lax.fori_loop(0, EPT // BD, chunk1, 0)
    plsc.subcore_barrier()

    pltpu.sync_copy(acc_sh.at[stripe], dz)

    def newton(g, _):
        sl = pl.ds(g * 16, 16)
        dz[sl] = _rsqrt16(dz[sl])
        return 0
    lax.fori_loop(0, SN // 16, newton, 0, unroll=4)
    pltpu.sync_copy(dz, acc_sh.at[stripe])
    plsc.subcore_barrier()

    pltpu.sync_copy(acc_sh, dinv_v)
    s_v = p1_v
    d_v = p3_v
    a_v = p2_v
    o_v = p4_v

    def chunk3(j, _):
        base = wid * EPW + j * BW
        pltpu.sync_copy(src_hbm.at[pl.ds(base, BW)], s_v)
        pltpu.sync_copy(dst_hbm.at[pl.ds(base, BW)], d_v)
        pltpu.sync_copy(ea_hbm.at[pl.ds(base, BW)], a_v)

        def grp(g, _):
            sl = pl.ds(g * 16, 16)
            ds_i = plsc.load_gather(dinv_v, [s_v[sl]])
            dd_i = plsc.load_gather(dinv_v, [d_v[sl]])
            o_v[sl] = a_v[sl] * ds_i * dd_i
            return 0
        lax.fori_loop(0, BW // 16, grp, 0, unroll=4)
        pltpu.sync_copy(o_v, w_hbm.at[pl.ds(base, BW)])
        return 0
    lax.fori_loop(0, EPW // BW, chunk3, 0)


_eprep_call = pl.kernel(
    _eprep_body,
    out_type=jax.ShapeDtypeStruct((EE,), _f32),
    mesh=_mesh,
    compiler_params=_sc_params,
    scratch_types=[
        pltpu.VMEM((NP,), _f32),
        pltpu.VMEM((BD,), _i32),
        pltpu.VMEM((BD,), _f32),
        pltpu.VMEM((BW,), _i32),
        pltpu.VMEM((BW,), _f32),
        pltpu.VMEM_SHARED((NP,), _f32),
    ],
)


KTAPS = 4


def _layer_body(z_hbm, src_hbm, dst_hbm, w_hbm, zn_hbm,
                s_v, d_v, w_v, rows_v, acc_sh,
                sin0, sin1, sg0, sg1, ss0, ss1):
    c = lax.axis_index("c")
    s = lax.axis_index("s")
    coff = c * NP
    sins = (sin0, sin1)
    sgs = (sg0, sg1)
    sss = (ss0, ss1)
    nch = EPT // BP
    ebase = s * EPT
    row0 = s * SN
    lane_consts = [jnp.full((16,), j, _i32) for j in range(16)]

    def zero_stripe():
        rv0 = rows_v.at[0]

        def zrow(i, _):
            rv0[i] = jnp.zeros((HH,), _f32)
            return 0
        lax.fori_loop(0, BP, zrow, 0)
        for k in range(SN // BP):
            pltpu.sync_copy(rv0, acc_sh.at[pl.ds(row0 + k * BP, BP)])
        _tail = SN - (SN // BP) * BP
        if _tail:
            pltpu.sync_copy(rv0.at[pl.ds(0, _tail)],
                            acc_sh.at[pl.ds(row0 + (SN // BP) * BP, _tail)])

    def in_copies(b, bd, j):
        base = ebase + j * BP
        return (pltpu.make_async_copy(src_hbm.at[pl.ds(base, BP)],
                                      s_v.at[b], sins[b]),
                pltpu.make_async_copy(dst_hbm.at[pl.ds(base, BP)],
                                      d_v.at[bd], sins[b]),
                pltpu.make_async_copy(w_hbm.at[pl.ds(base, BP)],
                                      w_v.at[b], sins[b]))

    def start_in(b, bd, j):
        for cp in in_copies(b, bd, j):
            cp.start()

    def wait_in(b, bd, j):
        for cp in in_copies(b, bd, j):
            cp.wait()

    def wait_scatter(b, bd):
        pltpu.make_async_copy(rows_v.at[b],
                              acc_sh.at[d_v.at[bd]], sss[b]).wait()

    def adj(b, goff):
        svb = s_v.at[b]

        def grp(g, _):
            sl = pl.ds(g * 16, 16)
            svb[sl] = svb[sl] + goff
            return 0
        lax.fori_loop(0, BP // 16, grp, 0, unroll=4)

    def scale(b):
        rvb = rows_v.at[b]
        wvb = w_v.at[b]

        def grp(g, _):
            wv = wvb[pl.ds(g * 16, 16)]
            r0 = g * 16
            for j in range(16):
                wj = wv.at[lane_consts[j]].get(mode="promise_in_bounds")
                rvb[r0 + j] = rvb[r0 + j] * wj
            return 0
        lax.fori_loop(0, BP // 16, grp, 0)

    def run_tap(gref, goff):
        def gather_copy(b):
            return pltpu.make_async_copy(gref.at[s_v.at[b]], rows_v.at[b],
                                         sgs[b])

        start_in(0, 0, 0)
        start_in(1, 1, 1)
        wait_in(0, 0, 0)
        adj(0, goff)
        gather_copy(0).start()

        def tenpack(t, _):
            for u in range(10):
                jj = 10 * t + u
                b = u % 2
                bo = 1 - b
                bd = u % 5
                bd1 = (u + 1) % 5
                bd2 = (u + 2) % 5

                @pl.when(jj + 1 < nch)
                def _():
                    wait_in(bo, bd1, jj + 1)
                    adj(bo, goff)

                @pl.when(jj >= 1)
                def _():
                    wait_scatter(bo, (u + 4) % 5)

                @pl.when(jj + 1 < nch)
                def _():
                    gather_copy(bo).start()
                gather_copy(b).wait()
                scale(b)

                @pl.when(jj + 2 < nch)
                def _():
                    start_in(b, bd2, jj + 2)
                pltpu.async_copy(rows_v.at[b], acc_sh.at[d_v.at[bd]],
                                 sss[b], add=True)
            return 0
        lax.fori_loop(0, nch // 10, tenpack, 0)
        wait_scatter(1, (nch - 1) % 5)

    zero_stripe()
    plsc.subcore_barrier()
    for t in range(KTAPS):
        if t == 0:
            run_tap(z_hbm, coff)
        else:
            run_tap(zn_hbm, (t - 1) * 2 * NP + coff)
        plsc.subcore_barrier()
        pltpu.sync_copy(acc_sh.at[pl.ds(row0, SN)],
                        zn_hbm.at[pl.ds(t * 2 * NP + coff + row0, SN)])
        if t < KTAPS - 1:
            zero_stripe()
        plsc.subcore_barrier()


_layer_call = pl.kernel(
    _layer_body,
    out_type=jax.ShapeDtypeStruct((KTAPS * 2 * NP, HH), _f32),
    mesh=_mesh,
    compiler_params=_sc_params,
    scratch_types=[
        pltpu.VMEM((2, BP), _i32),
        pltpu.VMEM((5, BP), _i32),
        pltpu.VMEM((2, BP), _f32),
        pltpu.VMEM((2, BP, HH), _f32),
        pltpu.VMEM_SHARED((NP, HH), _f32),
        pltpu.SemaphoreType.DMA,
        pltpu.SemaphoreType.DMA,
        pltpu.SemaphoreType.DMA,
        pltpu.SemaphoreType.DMA,
        pltpu.SemaphoreType.DMA,
        pltpu.SemaphoreType.DMA,
    ],
)


def _act(t):
    return jnp.where(t >= 0, t, 0.01 * t)


def _tc_in_body(state_ref, win_ref, bin_ref, xh_ref):
    x = _act(jnp.dot(state_ref[...], win_ref[...],
                     preferred_element_type=_f32) + bin_ref[...])
    xh_ref[0] = x[:, :HH]
    xh_ref[1] = x[:, HH:]


def _tc_in_call(state, W_in, b_in):
    return pl.pallas_call(
        _tc_in_body,
        grid=(NP // BLK,),
        in_specs=[
            pl.BlockSpec((BLK, DD), lambda i: (i, 0)),
            pl.BlockSpec((DD, CC), lambda i: (0, 0)),
            pl.BlockSpec((1, CC), lambda i: (0, 0)),
        ],
        out_specs=pl.BlockSpec((2, BLK, HH), lambda i: (0, i, 0)),
        out_shape=jax.ShapeDtypeStruct((2, NP, HH), _f32),
    )(state, W_in, b_in)


def _tc_layer_body(x0_ref, x1_ref, z10, z11, z20, z21, z30, z31, z40, z41,
                   W_ref, b_ref, out_ref):
    acc = b_ref[...]
    pairs = ((x0_ref, x1_ref), (z10, z11), (z20, z21), (z30, z31), (z40, z41))
    for k, (lo, hi) in enumerate(pairs):
        acc = acc + jnp.dot(lo[...], W_ref[k, :HH, :],
                            preferred_element_type=_f32)
        acc = acc + jnp.dot(hi[...], W_ref[k, HH:, :],
                            preferred_element_type=_f32)
    x = _act(acc)
    out_ref[0] = x[:, :HH]
    out_ref[1] = x[:, HH:]


def _half_specs(t=None):
    if t is None:
        return [pl.BlockSpec((BLK, HH), lambda i: (i, 0)),
                pl.BlockSpec((BLK, HH), lambda i: (i + NP // BLK, 0))]
    off = t * 2 * (NP // BLK)
    return [pl.BlockSpec((BLK, HH), lambda i, off=off: (i + off, 0)),
            pl.BlockSpec((BLK, HH),
                         lambda i, off=off: (i + off + NP // BLK, 0))]


def _tc_layer_call(xh, zst, Wl, bl):
    specs = _half_specs()
    for t in range(4):
        specs.extend(_half_specs(t))
    specs.append(pl.BlockSpec((5, CC, CC), lambda i: (0, 0, 0)))
    specs.append(pl.BlockSpec((1, CC), lambda i: (0, 0)))
    return pl.pallas_call(
        _tc_layer_body,
        grid=(NP // BLK,),
        in_specs=specs,
        out_specs=pl.BlockSpec((2, BLK, HH), lambda i: (0, i, 0)),
        out_shape=jax.ShapeDtypeStruct((2, NP, HH), _f32),
    )(xh, xh, zst, zst, zst, zst, zst, zst, zst, zst, Wl, bl)


def _tc_final_body(x0_ref, x1_ref, z10, z11, z20, z21, z30, z31, z40, z41,
                   W_ref, b_ref, wout_ref, bout_ref, y_ref):
    acc = b_ref[...]
    pairs = ((x0_ref, x1_ref), (z10, z11), (z20, z21), (z30, z31), (z40, z41))
    for k, (lo, hi) in enumerate(pairs):
        acc = acc + jnp.dot(lo[...], W_ref[k, :HH, :],
                            preferred_element_type=_f32)
        acc = acc + jnp.dot(hi[...], W_ref[k, HH:, :],
                            preferred_element_type=_f32)
    x = _act(acc)
    y_ref[...] = jnp.dot(x, wout_ref[...],
                         preferred_element_type=_f32) + bout_ref[...]


def _tc_final_call(xh, zst, Wl, bl, W_out, b_out):
    specs = _half_specs()
    for t in range(4):
        specs.extend(_half_specs(t))
    specs.append(pl.BlockSpec((5, CC, CC), lambda i: (0, 0, 0)))
    specs.append(pl.BlockSpec((1, CC), lambda i: (0, 0)))
    specs.append(pl.BlockSpec((CC, 1), lambda i: (0, 0)))
    specs.append(pl.BlockSpec((1, 1), lambda i: (0, 0)))
    return pl.pallas_call(
        _tc_final_body,
        grid=(NP // BLK,),
        in_specs=specs,
        out_specs=pl.BlockSpec((BLK, 1), lambda i: (i, 0)),
        out_shape=jax.ShapeDtypeStruct((NP, 1), _f32),
    )(xh, xh, zst, zst, zst, zst, zst, zst, zst, zst, Wl, bl, W_out, b_out)


def kernel(state, edge_index, edge_attr, W_in, b_in, W_gnn, b_gnn, W_out, b_out):
    src = edge_index[0]
    dst = edge_index[1]

    w_norm = _eprep_call(src, dst, edge_attr)
    xh = _tc_in_call(state, W_in, b_in.reshape(1, CC))

    x = xh.reshape(2 * NP, HH)
    L = W_gnn.shape[0]
    for l in range(L):
        zst = _layer_call(x, src, dst, w_norm)
        Wl = W_gnn[l]
        bl = b_gnn[l].reshape(1, CC)
        if l < L - 1:
            x = _tc_layer_call(x, zst, Wl, bl).reshape(2 * NP, HH)
        else:
            y = _tc_final_call(x, zst, Wl, bl,
                               W_out.reshape(CC, 1), b_out.reshape(1, 1))
    return y[:NN, 0]


# speedup vs baseline: 1.0655x; 1.0655x over previous
import functools

import jax
import jax.numpy as jnp
from jax import lax
from jax.experimental import pallas as pl
from jax.experimental.pallas import tpu as pltpu
from jax.experimental.pallas import tpu_sc as plsc

NN = 100000
EE = 1600000
DD = 128
CC = 32
HH = 16

NC = 2
NS = 16
BLK = 1024
NP = 98 * BLK
SN = NP // NS
ZB = SN // 8

EPT = EE // NS
EPW = EE // (NC * NS)
BD = 2000
BW = 2000
BP = 400

_mesh = plsc.VectorSubcoreMesh(
    core_axis_name="c", subcore_axis_name="s", num_cores=NC, num_subcores=NS)
_sc_params = pltpu.CompilerParams(needs_layout_passes=False,
                                  use_tc_tiling_on_sc=False)

_f32 = jnp.float32
_i32 = jnp.int32


def _rsqrt16(d):
    x = d + 1e-12
    i = lax.bitcast_convert_type(x, _i32)
    i = 0x5F3759DF - lax.shift_right_logical(i, 1)
    y = lax.bitcast_convert_type(i, _f32)
    for _ in range(3):
        y = y * (1.5 - 0.5 * x * y * y)
    return jnp.where(d > 0, y, 0.0)


def _eprep_body(src_hbm, dst_hbm, ea_hbm, w_hbm,
                dinv_v, p1_v, p2_v, p3_v, p4_v, acc_sh):
    c = lax.axis_index("c")
    s = lax.axis_index("s")
    wid = c * NS + s
    stripe = pl.ds(s * SN, SN)

    dz = dinv_v.at[pl.ds(0, SN)]

    def zrow(i, _):
        dz[pl.ds(i * 16, 16)] = jnp.zeros((16,), _f32)
        return 0
    lax.fori_loop(0, SN // 16, zrow, 0)
    pltpu.sync_copy(dz, acc_sh.at[stripe])
    plsc.subcore_barrier()

    def chunk1(j, _):
        base = s * EPT + j * BD
        pltpu.sync_copy(dst_hbm.at[pl.ds(base, BD)], p1_v)
        pltpu.sync_copy(ea_hbm.at[pl.ds(base, BD)], p2_v)
        pltpu.sync_copy(p2_v, acc_sh.at[p1_v], add=True)
        return 0
    lax.fori_loop(0, EPT // BD, chunk1, 0)
    plsc.subcore_barrier()

    pltpu.sync_copy(acc_sh.at[stripe], dz)

    def newton(g, _):
        sl = pl.ds(g * 16, 16)
        dz[sl] = _rsqrt16(dz[sl])
        return 0
    lax.fori_loop(0, SN // 16, newton, 0, unroll=4)
    pltpu.sync_copy(dz, acc_sh.at[stripe])
    plsc.subcore_barrier()

    pltpu.sync_copy(acc_sh, dinv_v)
    s_v = p1_v
    d_v = p3_v
    a_v = p2_v
    o_v = p4_v

    def chunk3(j, _):
        base = wid * EPW + j * BW
        pltpu.sync_copy(src_hbm.at[pl.ds(base, BW)], s_v)
        pltpu.sync_copy(dst_hbm.at[pl.ds(base, BW)], d_v)
        pltpu.sync_copy(ea_hbm.at[pl.ds(base, BW)], a_v)

        def grp(g, _):
            sl = pl.ds(g * 16, 16)
            ds_i = plsc.load_gather(dinv_v, [s_v[sl]])
            dd_i = plsc.load_gather(dinv_v, [d_v[sl]])
            o_v[sl] = a_v[sl] * ds_i * dd_i
            return 0
        lax.fori_loop(0, BW // 16, grp, 0, unroll=4)
        pltpu.sync_copy(o_v, w_hbm.at[pl.ds(base, BW)])
        return 0
    lax.fori_loop(0, EPW // BW, chunk3, 0)


_eprep_call = pl.kernel(
    _eprep_body,
    out_type=jax.ShapeDtypeStruct((EE,), _f32),
    mesh=_mesh,
    compiler_params=_sc_params,
    scratch_types=[
        pltpu.VMEM((NP,), _f32),
        pltpu.VMEM((BD,), _i32),
        pltpu.VMEM((BD,), _f32),
        pltpu.VMEM((BW,), _i32),
        pltpu.VMEM((BW,), _f32),
        pltpu.VMEM_SHARED((NP,), _f32),
    ],
)


def _prop_body(z_hbm, src_hbm, dst_hbm, w_hbm, zn_hbm,
               s_v, d_v, w_v, rows_v, acc_sh,
               sin0, sin1, sg0, sg1, ss0, ss1):
    c = lax.axis_index("c")
    s = lax.axis_index("s")
    coff = c * NP
    sins = (sin0, sin1)
    sgs = (sg0, sg1)
    sss = (ss0, ss1)
    nch = EPT // BP
    ebase = s * EPT

    rv0 = rows_v.at[0]

    def zrow(i, _):
        rv0[i] = jnp.zeros((HH,), _f32)
        return 0
    lax.fori_loop(0, BP, zrow, 0)
    row0 = s * SN
    for k in range(SN // BP):
        pltpu.sync_copy(rv0, acc_sh.at[pl.ds(row0 + k * BP, BP)])
    _tail = SN - (SN // BP) * BP
    if _tail:
        pltpu.sync_copy(rv0.at[pl.ds(0, _tail)],
                        acc_sh.at[pl.ds(row0 + (SN // BP) * BP, _tail)])
    plsc.subcore_barrier()

    def in_copies(b, bd, j):
        base = ebase + j * BP
        return (pltpu.make_async_copy(src_hbm.at[pl.ds(base, BP)],
                                      s_v.at[b], sins[b]),
                pltpu.make_async_copy(dst_hbm.at[pl.ds(base, BP)],
                                      d_v.at[bd], sins[b]),
                pltpu.make_async_copy(w_hbm.at[pl.ds(base, BP)],
                                      w_v.at[b], sins[b]))

    def start_in(b, bd, j):
        for cp in in_copies(b, bd, j):
            cp.start()

    def wait_in(b, bd, j):
        for cp in in_copies(b, bd, j):
            cp.wait()

    def gather_copy(b):
        return pltpu.make_async_copy(z_hbm.at[s_v.at[b]], rows_v.at[b], sgs[b])

    def wait_scatter(b, bd):
        pltpu.make_async_copy(rows_v.at[b],
                              acc_sh.at[d_v.at[bd]], sss[b]).wait()

    def adj(b):
        svb = s_v.at[b]

        def grp(g, _):
            sl = pl.ds(g * 16, 16)
            svb[sl] = svb[sl] + coff
            return 0
        lax.fori_loop(0, BP // 16, grp, 0, unroll=4)

    lane_consts = [jnp.full((16,), j, _i32) for j in range(16)]

    def scale(b):
        rvb = rows_v.at[b]
        wvb = w_v.at[b]

        def grp(g, _):
            wv = wvb[pl.ds(g * 16, 16)]
            r0 = g * 16
            for j in range(16):
                wj = wv.at[lane_consts[j]].get(mode="promise_in_bounds")
                rvb[r0 + j] = rvb[r0 + j] * wj
            return 0
        lax.fori_loop(0, BP // 16, grp, 0)

    start_in(0, 0, 0)
    start_in(1, 1, 1)
    wait_in(0, 0, 0)
    adj(0)
    gather_copy(0).start()

    def tenpack(t, _):
        for u in range(10):
            jj = 10 * t + u
            b = u % 2
            bo = 1 - b
            bd = u % 5
            bd1 = (u + 1) % 5
            bd2 = (u + 2) % 5

            @pl.when(jj + 1 < nch)
            def _():
                wait_in(bo, bd1, jj + 1)
                adj(bo)

            @pl.when(jj >= 1)
            def _():
                wait_scatter(bo, (u + 4) % 5)

            @pl.when(jj + 1 < nch)
            def _():
                gather_copy(bo).start()
            gather_copy(b).wait()
            scale(b)

            @pl.when(jj + 2 < nch)
            def _():
                start_in(b, bd2, jj + 2)
            pltpu.async_copy(rows_v.at[b], acc_sh.at[d_v.at[bd]], sss[b],
                             add=True)
        return 0
    lax.fori_loop(0, nch // 10, tenpack, 0)
    wait_scatter(1, (nch - 1) % 5)
    plsc.subcore_barrier()
    pltpu.sync_copy(acc_sh.at[pl.ds(s * SN, SN)],
                    zn_hbm.at[pl.ds(coff + s * SN, SN)])


_prop_call = pl.kernel(
    _prop_body,
    out_type=jax.ShapeDtypeStruct((2 * NP, HH), _f32),
    mesh=_mesh,
    compiler_params=_sc_params,
    scratch_types=[
        pltpu.VMEM((2, BP), _i32),
        pltpu.VMEM((5, BP), _i32),
        pltpu.VMEM((2, BP), _f32),
        pltpu.VMEM((2, BP, HH), _f32),
        pltpu.VMEM_SHARED((NP, HH), _f32),
        pltpu.SemaphoreType.DMA,
        pltpu.SemaphoreType.DMA,
        pltpu.SemaphoreType.DMA,
        pltpu.SemaphoreType.DMA,
        pltpu.SemaphoreType.DMA,
        pltpu.SemaphoreType.DMA,
    ],
)


def _act(t):
    return jnp.where(t >= 0, t, 0.01 * t)


def _tc_in_body(state_ref, win_ref, bin_ref, xh_ref):
    x = _act(jnp.dot(state_ref[...], win_ref[...],
                     preferred_element_type=_f32) + bin_ref[...])
    xh_ref[0] = x[:, :HH]
    xh_ref[1] = x[:, HH:]


def _tc_in_call(state, W_in, b_in):
    return pl.pallas_call(
        _tc_in_body,
        grid=(NP // BLK,),
        in_specs=[
            pl.BlockSpec((BLK, DD), lambda i: (i, 0)),
            pl.BlockSpec((DD, CC), lambda i: (0, 0)),
            pl.BlockSpec((1, CC), lambda i: (0, 0)),
        ],
        out_specs=pl.BlockSpec((2, BLK, HH), lambda i: (0, i, 0)),
        out_shape=jax.ShapeDtypeStruct((2, NP, HH), _f32),
    )(state, W_in, b_in)


def _tc_layer_body(x0_ref, x1_ref, z10, z11, z20, z21, z30, z31, z40, z41,
                   W_ref, b_ref, out_ref):
    acc = b_ref[...]
    pairs = ((x0_ref, x1_ref), (z10, z11), (z20, z21), (z30, z31), (z40, z41))
    for k, (lo, hi) in enumerate(pairs):
        acc = acc + jnp.dot(lo[...], W_ref[k, :HH, :],
                            preferred_element_type=_f32)
        acc = acc + jnp.dot(hi[...], W_ref[k, HH:, :],
                            preferred_element_type=_f32)
    x = _act(acc)
    out_ref[0] = x[:, :HH]
    out_ref[1] = x[:, HH:]


def _half_specs():
    return [pl.BlockSpec((BLK, HH), lambda i: (i, 0)),
            pl.BlockSpec((BLK, HH), lambda i: (i + NP // BLK, 0))]


def _tc_layer_call(xh, z1, z2, z3, z4, Wl, bl):
    specs = []
    for _ in range(5):
        specs.extend(_half_specs())
    specs.append(pl.BlockSpec((5, CC, CC), lambda i: (0, 0, 0)))
    specs.append(pl.BlockSpec((1, CC), lambda i: (0, 0)))
    return pl.pallas_call(
        _tc_layer_body,
        grid=(NP // BLK,),
        in_specs=specs,
        out_specs=pl.BlockSpec((2, BLK, HH), lambda i: (0, i, 0)),
        out_shape=jax.ShapeDtypeStruct((2, NP, HH), _f32),
    )(xh, xh, z1, z1, z2, z2, z3, z3, z4, z4, Wl, bl)


def _tc_final_body(x0_ref, x1_ref, z10, z11, z20, z21, z30, z31, z40, z41,
                   W_ref, b_ref, wout_ref, bout_ref, y_ref):
    acc = b_ref[...]
    pairs = ((x0_ref, x1_ref), (z10, z11), (z20, z21), (z30, z31), (z40, z41))
    for k, (lo, hi) in enumerate(pairs):
        acc = acc + jnp.dot(lo[...], W_ref[k, :HH, :],
                            preferred_element_type=_f32)
        acc = acc + jnp.dot(hi[...], W_ref[k, HH:, :],
                            preferred_element_type=_f32)
    x = _act(acc)
    y_ref[...] = jnp.dot(x, wout_ref[...],
                         preferred_element_type=_f32) + bout_ref[...]


def _tc_final_call(xh, z1, z2, z3, z4, Wl, bl, W_out, b_out):
    specs = []
    for _ in range(5):
        specs.extend(_half_specs())
    specs.append(pl.BlockSpec((5, CC, CC), lambda i: (0, 0, 0)))
    specs.append(pl.BlockSpec((1, CC), lambda i: (0, 0)))
    specs.append(pl.BlockSpec((CC, 1), lambda i: (0, 0)))
    specs.append(pl.BlockSpec((1, 1), lambda i: (0, 0)))
    return pl.pallas_call(
        _tc_final_body,
        grid=(NP // BLK,),
        in_specs=specs,
        out_specs=pl.BlockSpec((BLK, 1), lambda i: (i, 0)),
        out_shape=jax.ShapeDtypeStruct((NP, 1), _f32),
    )(xh, xh, z1, z1, z2, z2, z3, z3, z4, z4, Wl, bl, W_out, b_out)


def kernel(state, edge_index, edge_attr, W_in, b_in, W_gnn, b_gnn, W_out, b_out):
    src = edge_index[0]
    dst = edge_index[1]

    w_norm = _eprep_call(src, dst, edge_attr)
    xh = _tc_in_call(state, W_in, b_in.reshape(1, CC))

    x = xh.reshape(2 * NP, HH)
    L = W_gnn.shape[0]
    K = W_gnn.shape[1] - 1
    for l in range(L):
        zs = []
        z = x
        for _ in range(K):
            z = _prop_call(z, src, dst, w_norm)
            zs.append(z)
        Wl = W_gnn[l]
        bl = b_gnn[l].reshape(1, CC)
        if l < L - 1:
            x = _tc_layer_call(x, *zs, Wl, bl).reshape(2 * NP, HH)
        else:
            y = _tc_final_call(x, *zs, Wl, bl,
                               W_out.reshape(CC, 1), b_out.reshape(1, 1))
    return y[:NN, 0]


# speedup vs baseline: 1.0840x; 1.0174x over previous
import functools

import jax
import jax.numpy as jnp
from jax import lax
from jax.experimental import pallas as pl
from jax.experimental.pallas import tpu as pltpu
from jax.experimental.pallas import tpu_sc as plsc

NN = 100000
EE = 1600000
DD = 128
CC = 32
HH = 16

NC = 2
NS = 16
BLK = 1024
NP = 98 * BLK
SN = NP // NS
ZB = SN // 8

EPT = EE // NS
EPW = EE // (NC * NS)
BD = 2000
BW = 2000
BP = 400

_mesh = plsc.VectorSubcoreMesh(
    core_axis_name="c", subcore_axis_name="s", num_cores=NC, num_subcores=NS)
_sc_params = pltpu.CompilerParams(needs_layout_passes=False,
                                  use_tc_tiling_on_sc=False)

_f32 = jnp.float32
_i32 = jnp.int32


def _rsqrt16(d):
    x = d + 1e-12
    i = lax.bitcast_convert_type(x, _i32)
    i = 0x5F3759DF - lax.shift_right_logical(i, 1)
    y = lax.bitcast_convert_type(i, _f32)
    for _ in range(3):
        y = y * (1.5 - 0.5 * x * y * y)
    return jnp.where(d > 0, y, 0.0)


def _eprep_body(src_hbm, dst_hbm, ea_hbm, w_hbm,
                dinv_v, p1_v, p2_v, p3_v, p4_v, acc_sh,
                se0, se1, sw0, sw1):
    c = lax.axis_index("c")
    s = lax.axis_index("s")
    wid = c * NS + s
    stripe = pl.ds(s * SN, SN)
    ses = (se0, se1)
    sws = (sw0, sw1)

    dz = dinv_v.at[pl.ds(0, SN)]

    def zrow(i, _):
        dz[pl.ds(i * 16, 16)] = jnp.zeros((16,), _f32)
        return 0
    lax.fori_loop(0, SN // 16, zrow, 0)
    pltpu.sync_copy(dz, acc_sh.at[stripe])
    plsc.subcore_barrier()

    nch1 = EPT // BD

    def in1(b, j):
        base = s * EPT + j * BD
        return (pltpu.make_async_copy(dst_hbm.at[pl.ds(base, BD)],
                                      p1_v.at[b], ses[b]),
                pltpu.make_async_copy(ea_hbm.at[pl.ds(base, BD)],
                                      p2_v.at[b], ses[b]))

    for cp in in1(0, 0):
        cp.start()

    def chunk1(t, _):
        for b in (0, 1):
            j = 2 * t + b
            bo = 1 - b
            for cp in in1(b, j):
                cp.wait()

            @pl.when(j + 1 < nch1)
            def _():
                for cp in in1(bo, j + 1):
                    cp.start()
            pltpu.sync_copy(p2_v.at[b], acc_sh.at[p1_v.at[b]], add=True)
        return 0
    lax.fori_loop(0, nch1 // 2, chunk1, 0)
    plsc.subcore_barrier()

    pltpu.sync_copy(acc_sh.at[stripe], dz)

    def newton(g, _):
        sl = pl.ds(g * 16, 16)
        dz[sl] = _rsqrt16(dz[sl])
        return 0
    lax.fori_loop(0, SN // 16, newton, 0, unroll=4)
    pltpu.sync_copy(dz, acc_sh.at[stripe])
    plsc.subcore_barrier()

    pltpu.sync_copy(acc_sh, dinv_v)
    nch3 = EPW // BW

    def in3(b, j):
        base = wid * EPW + j * BW
        return (pltpu.make_async_copy(src_hbm.at[pl.ds(base, BW)],
                                      p1_v.at[b], ses[b]),
                pltpu.make_async_copy(dst_hbm.at[pl.ds(base, BW)],
                                      p3_v.at[b], ses[b]),
                pltpu.make_async_copy(ea_hbm.at[pl.ds(base, BW)],
                                      p2_v.at[b], ses[b]))

    def out3(b, j):
        base = wid * EPW + j * BW
        return pltpu.make_async_copy(p4_v.at[b],
                                     w_hbm.at[pl.ds(base, BW)], sws[b])

    def wcompute(b):
        s_v = p1_v.at[b]
        d_v = p3_v.at[b]
        a_v = p2_v.at[b]
        o_v = p4_v.at[b]

        def grp(g, _):
            sl = pl.ds(g * 16, 16)
            ds_i = plsc.load_gather(dinv_v, [s_v[sl]])
            dd_i = plsc.load_gather(dinv_v, [d_v[sl]])
            o_v[sl] = a_v[sl] * ds_i * dd_i
            return 0
        lax.fori_loop(0, BW // 16, grp, 0, unroll=4)

    for cp in in3(0, 0):
        cp.start()

    def chunk3(t, _):
        for b in (0, 1):
            j = 2 * t + b
            bo = 1 - b
            for cp in in3(b, j):
                cp.wait()

            @pl.when(j >= 1)
            def _():
                out3(bo, j - 1).wait()

            @pl.when(j + 1 < nch3)
            def _():
                for cp in in3(bo, j + 1):
                    cp.start()
            wcompute(b)
            out3(b, j).start()
        return 0
    lax.fori_loop(0, nch3 // 2, chunk3, 0)
    jl = nch3 - 1
    for cp in in3(0, jl):
        cp.wait()
    out3(1, jl - 1).wait()
    wcompute(0)
    out3(0, jl).start()
    out3(0, jl).wait()


_eprep_call = pl.kernel(
    _eprep_body,
    out_type=jax.ShapeDtypeStruct((EE,), _f32),
    mesh=_mesh,
    compiler_params=_sc_params,
    scratch_types=[
        pltpu.VMEM((NP,), _f32),
        pltpu.VMEM((2, BW), _i32),
        pltpu.VMEM((2, BW), _f32),
        pltpu.VMEM((2, BW), _i32),
        pltpu.VMEM((2, BW), _f32),
        pltpu.VMEM_SHARED((NP,), _f32),
        pltpu.SemaphoreType.DMA,
        pltpu.SemaphoreType.DMA,
        pltpu.SemaphoreType.DMA,
        pltpu.SemaphoreType.DMA,
    ],
)


def _prop_body(z_hbm, src_hbm, dst_hbm, w_hbm, zn_hbm,
               s_v, d_v, w_v, rows_v, acc_sh,
               sin0, sin1, sg0, sg1, ss0, ss1):
    c = lax.axis_index("c")
    s = lax.axis_index("s")
    coff = c * NP
    sins = (sin0, sin1)
    sgs = (sg0, sg1)
    sss = (ss0, ss1)
    nch = EPT // BP
    ebase = s * EPT

    rv0 = rows_v.at[0]

    def zrow(i, _):
        rv0[i] = jnp.zeros((HH,), _f32)
        return 0
    lax.fori_loop(0, BP, zrow, 0)
    row0 = s * SN
    for k in range(SN // BP):
        pltpu.sync_copy(rv0, acc_sh.at[pl.ds(row0 + k * BP, BP)])
    _tail = SN - (SN // BP) * BP
    if _tail:
        pltpu.sync_copy(rv0.at[pl.ds(0, _tail)],
                        acc_sh.at[pl.ds(row0 + (SN // BP) * BP, _tail)])
    plsc.subcore_barrier()

    def in_copies(b, bd, j):
        base = ebase + j * BP
        return (pltpu.make_async_copy(src_hbm.at[pl.ds(base, BP)],
                                      s_v.at[b], sins[b]),
                pltpu.make_async_copy(dst_hbm.at[pl.ds(base, BP)],
                                      d_v.at[bd], sins[b]),
                pltpu.make_async_copy(w_hbm.at[pl.ds(base, BP)],
                                      w_v.at[b], sins[b]))

    def start_in(b, bd, j):
        for cp in in_copies(b, bd, j):
            cp.start()

    def wait_in(b, bd, j):
        for cp in in_copies(b, bd, j):
            cp.wait()

    def gather_copy(b):
        return pltpu.make_async_copy(z_hbm.at[s_v.at[b]], rows_v.at[b], sgs[b])

    def wait_scatter(b, bd):
        pltpu.make_async_copy(rows_v.at[b],
                              acc_sh.at[d_v.at[bd]], sss[b]).wait()

    def adj(b):
        svb = s_v.at[b]

        def grp(g, _):
            sl = pl.ds(g * 16, 16)
            svb[sl] = svb[sl] + coff
            return 0
        lax.fori_loop(0, BP // 16, grp, 0, unroll=4)

    lane_consts = [jnp.full((16,), j, _i32) for j in range(16)]

    def scale(b):
        rvb = rows_v.at[b]
        wvb = w_v.at[b]

        def grp(g, _):
            wv = wvb[pl.ds(g * 16, 16)]
            r0 = g * 16
            for j in range(16):
                wj = wv.at[lane_consts[j]].get(mode="promise_in_bounds")
                rvb[r0 + j] = rvb[r0 + j] * wj
            return 0
        lax.fori_loop(0, BP // 16, grp, 0)

    start_in(0, 0, 0)
    start_in(1, 1, 1)
    wait_in(0, 0, 0)
    adj(0)
    gather_copy(0).start()

    def tenpack(t, _):
        for u in range(10):
            jj = 10 * t + u
            b = u % 2
            bo = 1 - b
            bd = u % 5
            bd1 = (u + 1) % 5
            bd2 = (u + 2) % 5

            @pl.when(jj + 1 < nch)
            def _():
                wait_in(bo, bd1, jj + 1)
                adj(bo)

            @pl.when(jj >= 1)
            def _():
                wait_scatter(bo, (u + 4) % 5)

            @pl.when(jj + 1 < nch)
            def _():
                gather_copy(bo).start()
            gather_copy(b).wait()
            scale(b)

            @pl.when(jj + 2 < nch)
            def _():
                start_in(b, bd2, jj + 2)
            pltpu.async_copy(rows_v.at[b], acc_sh.at[d_v.at[bd]], sss[b],
                             add=True)
        return 0
    lax.fori_loop(0, nch // 10, tenpack, 0)
    wait_scatter(1, (nch - 1) % 5)
    plsc.subcore_barrier()
    pltpu.sync_copy(acc_sh.at[pl.ds(s * SN, SN)],
                    zn_hbm.at[pl.ds(coff + s * SN, SN)])


_prop_call = pl.kernel(
    _prop_body,
    out_type=jax.ShapeDtypeStruct((2 * NP, HH), _f32),
    mesh=_mesh,
    compiler_params=_sc_params,
    scratch_types=[
        pltpu.VMEM((2, BP), _i32),
        pltpu.VMEM((5, BP), _i32),
        pltpu.VMEM((2, BP), _f32),
        pltpu.VMEM((2, BP, HH), _f32),
        pltpu.VMEM_SHARED((NP, HH), _f32),
        pltpu.SemaphoreType.DMA,
        pltpu.SemaphoreType.DMA,
        pltpu.SemaphoreType.DMA,
        pltpu.SemaphoreType.DMA,
        pltpu.SemaphoreType.DMA,
        pltpu.SemaphoreType.DMA,
    ],
)


def _act(t):
    return jnp.where(t >= 0, t, 0.01 * t)


def _tc_in_body(state_ref, win_ref, bin_ref, xh_ref):
    x = _act(jnp.dot(state_ref[...], win_ref[...],
                     preferred_element_type=_f32) + bin_ref[...])
    xh_ref[0] = x[:, :HH]
    xh_ref[1] = x[:, HH:]


def _tc_in_call(state, W_in, b_in):
    return pl.pallas_call(
        _tc_in_body,
        grid=(NP // BLK,),
        in_specs=[
            pl.BlockSpec((BLK, DD), lambda i: (i, 0)),
            pl.BlockSpec((DD, CC), lambda i: (0, 0)),
            pl.BlockSpec((1, CC), lambda i: (0, 0)),
        ],
        out_specs=pl.BlockSpec((2, BLK, HH), lambda i: (0, i, 0)),
        out_shape=jax.ShapeDtypeStruct((2, NP, HH), _f32),
    )(state, W_in, b_in)


def _tc_layer_body(x0_ref, x1_ref, z10, z11, z20, z21, z30, z31, z40, z41,
                   W_ref, b_ref, out_ref):
    acc = b_ref[...]
    pairs = ((x0_ref, x1_ref), (z10, z11), (z20, z21), (z30, z31), (z40, z41))
    for k, (lo, hi) in enumerate(pairs):
        acc = acc + jnp.dot(lo[...], W_ref[k, :HH, :],
                            preferred_element_type=_f32)
        acc = acc + jnp.dot(hi[...], W_ref[k, HH:, :],
                            preferred_element_type=_f32)
    x = _act(acc)
    out_ref[0] = x[:, :HH]
    out_ref[1] = x[:, HH:]


def _half_specs():
    return [pl.BlockSpec((BLK, HH), lambda i: (i, 0)),
            pl.BlockSpec((BLK, HH), lambda i: (i + NP // BLK, 0))]


def _tc_layer_call(xh, z1, z2, z3, z4, Wl, bl):
    specs = []
    for _ in range(5):
        specs.extend(_half_specs())
    specs.append(pl.BlockSpec((5, CC, CC), lambda i: (0, 0, 0)))
    specs.append(pl.BlockSpec((1, CC), lambda i: (0, 0)))
    return pl.pallas_call(
        _tc_layer_body,
        grid=(NP // BLK,),
        in_specs=specs,
        out_specs=pl.BlockSpec((2, BLK, HH), lambda i: (0, i, 0)),
        out_shape=jax.ShapeDtypeStruct((2, NP, HH), _f32),
    )(xh, xh, z1, z1, z2, z2, z3, z3, z4, z4, Wl, bl)


def _tc_final_body(x0_ref, x1_ref, z10, z11, z20, z21, z30, z31, z40, z41,
                   W_ref, b_ref, wout_ref, bout_ref, y_ref):
    acc = b_ref[...]
    pairs = ((x0_ref, x1_ref), (z10, z11), (z20, z21), (z30, z31), (z40, z41))
    for k, (lo, hi) in enumerate(pairs):
        acc = acc + jnp.dot(lo[...], W_ref[k, :HH, :],
                            preferred_element_type=_f32)
        acc = acc + jnp.dot(hi[...], W_ref[k, HH:, :],
                            preferred_element_type=_f32)
    x = _act(acc)
    y_ref[...] = jnp.dot(x, wout_ref[...],
                         preferred_element_type=_f32) + bout_ref[...]


def _tc_final_call(xh, z1, z2, z3, z4, Wl, bl, W_out, b_out):
    specs = []
    for _ in range(5):
        specs.extend(_half_specs())
    specs.append(pl.BlockSpec((5, CC, CC), lambda i: (0, 0, 0)))
    specs.append(pl.BlockSpec((1, CC), lambda i: (0, 0)))
    specs.append(pl.BlockSpec((CC, 1), lambda i: (0, 0)))
    specs.append(pl.BlockSpec((1, 1), lambda i: (0, 0)))
    return pl.pallas_call(
        _tc_final_body,
        grid=(NP // BLK,),
        in_specs=specs,
        out_specs=pl.BlockSpec((BLK, 1), lambda i: (i, 0)),
        out_shape=jax.ShapeDtypeStruct((NP, 1), _f32),
    )(xh, xh, z1, z1, z2, z2, z3, z3, z4, z4, Wl, bl, W_out, b_out)


def kernel(state, edge_index, edge_attr, W_in, b_in, W_gnn, b_gnn, W_out, b_out):
    src = edge_index[0]
    dst = edge_index[1]

    w_norm = _eprep_call(src, dst, edge_attr)
    xh = _tc_in_call(state, W_in, b_in.reshape(1, CC))

    x = xh.reshape(2 * NP, HH)
    L = W_gnn.shape[0]
    K = W_gnn.shape[1] - 1
    for l in range(L):
        zs = []
        z = x
        for _ in range(K):
            z = _prop_call(z, src, dst, w_norm)
            zs.append(z)
        Wl = W_gnn[l]
        bl = b_gnn[l].reshape(1, CC)
        if l < L - 1:
            x = _tc_layer_call(x, *zs, Wl, bl).reshape(2 * NP, HH)
        else:
            y = _tc_final_call(x, *zs, Wl, bl,
                               W_out.reshape(CC, 1), b_out.reshape(1, 1))
    return y[:NN, 0]
